# baseline (speedup 1.0000x reference)
import jax
import jax.numpy as jnp
from jax import lax
from jax.experimental import pallas as pl
from jax.experimental.pallas import tpu as pltpu

_STEPS = 5
_BATCH = 1024
_N_ITEMS = 100000
_N_TOTAL = _BATCH * _N_ITEMS

_NOISE_KEY = (0x85F65B85, 0x97B8C3E1)
_BERN_KEY = (0x181B3F15, 0x67A69C51)

_ROTS = ((13, 15, 26, 6), (17, 29, 16, 24))

_RB1 = 8
_RB2 = 128
_CB2 = 2048
_GC2 = (_N_ITEMS + _CB2 - 1) // _CB2


def _rotl(x, d):
    return lax.shift_left(x, jnp.uint32(d)) | lax.shift_right_logical(
        x, jnp.uint32(32 - d)
    )


def _threefry_bits(key, lo):
    k1 = jnp.uint32(key[0])
    k2 = jnp.uint32(key[1])
    k3 = k1 ^ k2 ^ jnp.uint32(0x1BD11BDA)
    ks = (k1, k2, k3)
    x0 = jnp.full_like(lo, k1)
    x1 = lo + k2
    for g in range(5):
        for r in _ROTS[g % 2]:
            x0 = x0 + x1
            x1 = _rotl(x1, r) ^ x0
        x0 = x0 + ks[(g + 1) % 3]
        x1 = x1 + ks[(g + 2) % 3] + jnp.uint32(g + 1)
    return x0 ^ x1


def _u01(bits):
    fb = lax.shift_right_logical(bits, jnp.uint32(9)) | jnp.uint32(0x3F800000)
    return lax.bitcast_convert_type(fb, jnp.float32) - 1.0


def _count_zero_kernel(x_ref, cnt_ref):
    @pl.when(pl.program_id(0) == 0)
    def _init():
        cnt_ref[0, 0] = jnp.int32(0)

    blk = x_ref[...]
    cnt_ref[0, 0] += jnp.sum((blk == 0.0).astype(jnp.int32))


def _sample_kernel(x_ref, a0_ref, a1_ref, out_ref):
    r0 = pl.program_id(0) * _RB2
    c0 = pl.program_id(1) * _CB2
    rows = lax.broadcasted_iota(jnp.int32, (_RB2, _CB2), 0)
    cols = lax.broadcasted_iota(jnp.int32, (_RB2, _CB2), 1)
    lin = ((r0 + rows) * _N_ITEMS + (c0 + cols)).astype(jnp.uint32)

    x = x_ref[...]
    u_n = _u01(_threefry_bits(_NOISE_KEY, lin))
    u_b = _u01(_threefry_bits(_BERN_KEY, lin))
    a = jnp.where(x == 0.0, a0_ref[...], a1_ref[...])
    p = jax.nn.sigmoid(a - u_n)
    flip = u_b < p
    out_ref[...] = jnp.where(flip, 1.0 - x, x)


def kernel(x_start, t):
    cnt = pl.pallas_call(
        _count_zero_kernel,
        grid=(_BATCH // _RB1,),
        in_specs=[pl.BlockSpec((_RB1, _N_ITEMS), lambda i: (i, 0))],
        out_specs=pl.BlockSpec(
            (1, 1), lambda i: (0, 0), memory_space=pltpu.SMEM
        ),
        out_shape=jax.ShapeDtypeStruct((1, 1), jnp.int32),
        compiler_params=pltpu.CompilerParams(
            dimension_semantics=("arbitrary",)
        ),
    )(x_start)

    sparsity = cnt[0, 0].astype(jnp.float32) / jnp.float32(_N_TOTAL)
    gamma_start = 0.1 * (1.0 - sparsity) + 0.001
    gamma_end = gamma_start * 0.1
    epsilon_start = 0.005 * sparsity + 0.0001
    epsilon_end = epsilon_start * 0.1
    gamma = jnp.linspace(gamma_start, gamma_end, _STEPS)
    epsilon = jnp.linspace(epsilon_start, epsilon_end, _STEPS)
    epsilon = jnp.minimum(epsilon, 0.01)
    gamma_cum = 1.0 - jnp.cumprod(1.0 - gamma)
    epsilon_cum = 1.0 - jnp.cumprod(1.0 - epsilon)

    a0 = jnp.take(gamma_cum, t, axis=0)[:, None]
    a1 = jnp.take(epsilon_cum, t, axis=0)[:, None]

    return pl.pallas_call(
        _sample_kernel,
        grid=(_BATCH // _RB2, _GC2),
        in_specs=[
            pl.BlockSpec((_RB2, _CB2), lambda i, j: (i, j)),
            pl.BlockSpec((_RB2, 1), lambda i, j: (i, 0)),
            pl.BlockSpec((_RB2, 1), lambda i, j: (i, 0)),
        ],
        out_specs=pl.BlockSpec((_RB2, _CB2), lambda i, j: (i, j)),
        out_shape=jax.ShapeDtypeStruct((_BATCH, _N_ITEMS), jnp.float32),
        compiler_params=pltpu.CompilerParams(
            dimension_semantics=("parallel", "parallel")
        ),
    )(x_start, a0, a1)


# speedup vs baseline: 1.4608x; 1.4608x over previous
import jax
import jax.numpy as jnp
from jax import lax
from jax.experimental import pallas as pl
from jax.experimental.pallas import tpu as pltpu

_STEPS = 5
_BATCH = 1024
_N_ITEMS = 100000
_N_TOTAL = _BATCH * _N_ITEMS

_NOISE_KEY = (0x85F65B85, 0x97B8C3E1)
_BERN_KEY = (0x181B3F15, 0x67A69C51)

_ROTS = ((13, 15, 26, 6), (17, 29, 16, 24))

_RB1 = 8
_RB2 = 128
_CB2 = 2048
_GC2 = (_N_ITEMS + _CB2 - 1) // _CB2


def _rotl(x, d):
    return lax.shift_left(x, jnp.uint32(d)) | lax.shift_right_logical(
        x, jnp.uint32(32 - d)
    )


def _threefry_bits(key, lo):
    k1 = jnp.uint32(key[0])
    k2 = jnp.uint32(key[1])
    k3 = k1 ^ k2 ^ jnp.uint32(0x1BD11BDA)
    ks = (k1, k2, k3)
    x0 = jnp.full_like(lo, k1)
    x1 = lo + k2
    for g in range(5):
        for r in _ROTS[g % 2]:
            x0 = x0 + x1
            x1 = _rotl(x1, r) ^ x0
        x0 = x0 + ks[(g + 1) % 3]
        x1 = x1 + ks[(g + 2) % 3] + jnp.uint32(g + 1)
    return x0 ^ x1


def _u01(bits):
    fb = lax.shift_right_logical(bits, jnp.uint32(9)) | jnp.uint32(0x3F800000)
    return lax.bitcast_convert_type(fb, jnp.float32) - 1.0


def _count_zero_kernel(x_ref, cnt_ref):
    @pl.when(pl.program_id(0) == 0)
    def _init():
        cnt_ref[0, 0] = jnp.int32(0)

    blk = x_ref[...]
    cnt_ref[0, 0] += jnp.sum((blk == 0.0).astype(jnp.int32))


_RCH = 8
_CCH = 512
_NRC = _RB2 // _RCH
_NCC = _CB2 // _CCH


def _sample_kernel(x_ref, a0_ref, a1_ref, out_ref):
    r0 = pl.program_id(0) * _RB2
    c0 = pl.program_id(1) * _CB2
    rows = lax.broadcasted_iota(jnp.int32, (_RCH, _CCH), 0)
    cols = lax.broadcasted_iota(jnp.int32, (_RCH, _CCH), 1)

    def body(k, _):
        rc = k % _NRC
        cc = k // _NRC
        rs = rc * _RCH
        cs = cc * _CCH
        lin = (
            (r0 + rs + rows) * _N_ITEMS + (c0 + cs + cols)
        ).astype(jnp.uint32)
        u_b = _u01(_threefry_bits(_BERN_KEY, lin))
        u_n = _u01(_threefry_bits(_NOISE_KEY, lin))
        x = x_ref[pl.ds(rs, _RCH), pl.ds(cs, _CCH)]
        a = jnp.where(x == 0.0, a0_ref[pl.ds(rs, _RCH), :], a1_ref[pl.ds(rs, _RCH), :])
        p = jax.nn.sigmoid(a - u_n)
        flip = u_b < p
        out_ref[pl.ds(rs, _RCH), pl.ds(cs, _CCH)] = jnp.where(flip, 1.0 - x, x)
        return _

    lax.fori_loop(0, _NRC * _NCC, body, 0)


def kernel(x_start, t):
    cnt = pl.pallas_call(
        _count_zero_kernel,
        grid=(_BATCH // _RB1,),
        in_specs=[pl.BlockSpec((_RB1, _N_ITEMS), lambda i: (i, 0))],
        out_specs=pl.BlockSpec(
            (1, 1), lambda i: (0, 0), memory_space=pltpu.SMEM
        ),
        out_shape=jax.ShapeDtypeStruct((1, 1), jnp.int32),
        compiler_params=pltpu.CompilerParams(
            dimension_semantics=("arbitrary",)
        ),
    )(x_start)

    sparsity = cnt[0, 0].astype(jnp.float32) / jnp.float32(_N_TOTAL)
    gamma_start = 0.1 * (1.0 - sparsity) + 0.001
    gamma_end = gamma_start * 0.1
    epsilon_start = 0.005 * sparsity + 0.0001
    epsilon_end = epsilon_start * 0.1
    gamma = jnp.linspace(gamma_start, gamma_end, _STEPS)
    epsilon = jnp.linspace(epsilon_start, epsilon_end, _STEPS)
    epsilon = jnp.minimum(epsilon, 0.01)
    gamma_cum = 1.0 - jnp.cumprod(1.0 - gamma)
    epsilon_cum = 1.0 - jnp.cumprod(1.0 - epsilon)

    a0 = jnp.take(gamma_cum, t, axis=0)[:, None]
    a1 = jnp.take(epsilon_cum, t, axis=0)[:, None]

    return pl.pallas_call(
        _sample_kernel,
        grid=(_BATCH // _RB2, _GC2),
        in_specs=[
            pl.BlockSpec((_RB2, _CB2), lambda i, j: (i, j)),
            pl.BlockSpec((_RB2, 1), lambda i, j: (i, 0)),
            pl.BlockSpec((_RB2, 1), lambda i, j: (i, 0)),
        ],
        out_specs=pl.BlockSpec((_RB2, _CB2), lambda i, j: (i, j)),
        out_shape=jax.ShapeDtypeStruct((_BATCH, _N_ITEMS), jnp.float32),
        compiler_params=pltpu.CompilerParams(
            dimension_semantics=("parallel", "parallel")
        ),
    )(x_start, a0, a1)


# speedup vs baseline: 1.5367x; 1.0519x over previous
import jax
import jax.numpy as jnp
from jax import lax
from jax.experimental import pallas as pl
from jax.experimental.pallas import tpu as pltpu

_STEPS = 5
_BATCH = 1024
_N_ITEMS = 100000
_N_TOTAL = _BATCH * _N_ITEMS

_NOISE_KEY = (0x85F65B85, 0x97B8C3E1)
_BERN_KEY = (0x181B3F15, 0x67A69C51)

_ROTS = ((13, 15, 26, 6), (17, 29, 16, 24))

_RB1 = 8
_RB2 = 128
_CB2 = 2048
_GC2 = (_N_ITEMS + _CB2 - 1) // _CB2


def _rotl(x, d):
    return lax.shift_left(x, jnp.uint32(d)) | lax.shift_right_logical(
        x, jnp.uint32(32 - d)
    )


def _threefry_bits(key, lo):
    k1 = jnp.uint32(key[0])
    k2 = jnp.uint32(key[1])
    k3 = k1 ^ k2 ^ jnp.uint32(0x1BD11BDA)
    ks = (k1, k2, k3)
    x0 = jnp.full_like(lo, k1)
    x1 = lo + k2
    for g in range(5):
        for r in _ROTS[g % 2]:
            x0 = x0 + x1
            x1 = _rotl(x1, r) ^ x0
        x0 = x0 + ks[(g + 1) % 3]
        x1 = x1 + ks[(g + 2) % 3] + jnp.uint32(g + 1)
    return x0 ^ x1


def _u01(bits):
    fb = lax.shift_right_logical(bits, jnp.uint32(9)) | jnp.uint32(0x3F800000)
    return lax.bitcast_convert_type(fb, jnp.float32) - 1.0


def _count_zero_kernel(x_ref, cnt_ref):
    @pl.when(pl.program_id(0) == 0)
    def _init():
        cnt_ref[0, 0] = jnp.int32(0)

    blk = x_ref[...]
    cnt_ref[0, 0] += jnp.sum((blk == 0.0).astype(jnp.int32))


_RCH = 8
_CCH = 512
_NRC = _RB2 // _RCH
_NCC = _CB2 // _CCH


def _sample_kernel(x_ref, a0_ref, a1_ref, out_ref):
    r0 = pl.program_id(0) * _RB2
    c0 = pl.program_id(1) * _CB2
    rows = lax.broadcasted_iota(jnp.int32, (_RCH, _CCH), 0)
    cols = lax.broadcasted_iota(jnp.int32, (_RCH, _CCH), 1)

    def body(k, _):
        rc = k & (_NRC - 1)
        cc = k >> _NRC.bit_length() - 1
        rs = rc * _RCH
        cs = cc * _CCH
        lin = (
            (r0 + rs + rows) * _N_ITEMS + (c0 + cs + cols)
        ).astype(jnp.uint32)
        u_b = _u01(_threefry_bits(_BERN_KEY, lin))
        u_n = _u01(_threefry_bits(_NOISE_KEY, lin))
        x = x_ref[pl.ds(rs, _RCH), pl.ds(cs, _CCH)]
        a = jnp.where(x == 0.0, a0_ref[pl.ds(rs, _RCH), :], a1_ref[pl.ds(rs, _RCH), :])
        p = jax.nn.sigmoid(a - u_n)
        flip = u_b < p
        out_ref[pl.ds(rs, _RCH), pl.ds(cs, _CCH)] = jnp.where(flip, 1.0 - x, x)
        return _

    lax.fori_loop(0, _NRC * _NCC, body, 0, unroll=2)


def kernel(x_start, t):
    cnt = pl.pallas_call(
        _count_zero_kernel,
        grid=(_BATCH // _RB1,),
        in_specs=[pl.BlockSpec((_RB1, _N_ITEMS), lambda i: (i, 0))],
        out_specs=pl.BlockSpec(
            (1, 1), lambda i: (0, 0), memory_space=pltpu.SMEM
        ),
        out_shape=jax.ShapeDtypeStruct((1, 1), jnp.int32),
        compiler_params=pltpu.CompilerParams(
            dimension_semantics=("arbitrary",)
        ),
    )(x_start)

    sparsity = cnt[0, 0].astype(jnp.float32) / jnp.float32(_N_TOTAL)
    gamma_start = 0.1 * (1.0 - sparsity) + 0.001
    gamma_end = gamma_start * 0.1
    epsilon_start = 0.005 * sparsity + 0.0001
    epsilon_end = epsilon_start * 0.1
    gamma = jnp.linspace(gamma_start, gamma_end, _STEPS)
    epsilon = jnp.linspace(epsilon_start, epsilon_end, _STEPS)
    epsilon = jnp.minimum(epsilon, 0.01)
    gamma_cum = 1.0 - jnp.cumprod(1.0 - gamma)
    epsilon_cum = 1.0 - jnp.cumprod(1.0 - epsilon)

    a0 = jnp.take(gamma_cum, t, axis=0)[:, None]
    a1 = jnp.take(epsilon_cum, t, axis=0)[:, None]

    return pl.pallas_call(
        _sample_kernel,
        grid=(_BATCH // _RB2, _GC2),
        in_specs=[
            pl.BlockSpec((_RB2, _CB2), lambda i, j: (i, j)),
            pl.BlockSpec((_RB2, 1), lambda i, j: (i, 0)),
            pl.BlockSpec((_RB2, 1), lambda i, j: (i, 0)),
        ],
        out_specs=pl.BlockSpec((_RB2, _CB2), lambda i, j: (i, j)),
        out_shape=jax.ShapeDtypeStruct((_BATCH, _N_ITEMS), jnp.float32),
        compiler_params=pltpu.CompilerParams(
            dimension_semantics=("parallel", "parallel")
        ),
    )(x_start, a0, a1)


# speedup vs baseline: 1.5628x; 1.0170x over previous
import jax
import jax.numpy as jnp
from jax import lax
from jax.experimental import pallas as pl
from jax.experimental.pallas import tpu as pltpu

_STEPS = 5
_BATCH = 1024
_N_ITEMS = 100000
_N_TOTAL = _BATCH * _N_ITEMS

_NOISE_KEY = (0x85F65B85, 0x97B8C3E1)
_BERN_KEY = (0x181B3F15, 0x67A69C51)

_ROTS = ((13, 15, 26, 6), (17, 29, 16, 24))

_RB1 = 8
_RB2 = 128
_CB2 = 2048
_GC2 = (_N_ITEMS + _CB2 - 1) // _CB2


def _rotl(x, d):
    return lax.shift_left(x, jnp.uint32(d)) | lax.shift_right_logical(
        x, jnp.uint32(32 - d)
    )


def _threefry_bits(key, lo):
    k1 = jnp.uint32(key[0])
    k2 = jnp.uint32(key[1])
    k3 = k1 ^ k2 ^ jnp.uint32(0x1BD11BDA)
    ks = (k1, k2, k3)
    x0 = jnp.full_like(lo, k1)
    x1 = lo + k2
    for g in range(5):
        for r in _ROTS[g % 2]:
            x0 = x0 + x1
            x1 = _rotl(x1, r) ^ x0
        x0 = x0 + ks[(g + 1) % 3]
        x1 = x1 + ks[(g + 2) % 3] + jnp.uint32(g + 1)
    return x0 ^ x1


def _u01(bits):
    fb = lax.shift_right_logical(bits, jnp.uint32(9)) | jnp.uint32(0x3F800000)
    return lax.bitcast_convert_type(fb, jnp.float32) - 1.0


def _count_zero_kernel(x_ref, cnt_ref):
    @pl.when(pl.program_id(0) == 0)
    def _init():
        cnt_ref[0, 0] = jnp.int32(0)

    blk = x_ref[...]
    cnt_ref[0, 0] += jnp.sum((blk == 0.0).astype(jnp.int32))


_RCH = 8
_CCH = 512
_NRC = _RB2 // _RCH
_NCC = _CB2 // _CCH


def _sample_kernel(x_ref, a0_ref, a1_ref, out_ref):
    r0 = pl.program_id(0) * _RB2
    c0 = pl.program_id(1) * _CB2
    rows = lax.broadcasted_iota(jnp.int32, (_RCH, _CCH), 0)
    cols = lax.broadcasted_iota(jnp.int32, (_RCH, _CCH), 1)

    def body(k, _):
        rc = k & (_NRC - 1)
        cc = k >> _NRC.bit_length() - 1
        rs = rc * _RCH
        cs = cc * _CCH
        lin = (
            (r0 + rs + rows) * _N_ITEMS + (c0 + cs + cols)
        ).astype(jnp.uint32)
        u_b = _u01(_threefry_bits(_BERN_KEY, lin))
        u_n = _u01(_threefry_bits(_NOISE_KEY, lin))
        x = x_ref[pl.ds(rs, _RCH), pl.ds(cs, _CCH)]
        a = jnp.where(x == 0.0, a0_ref[pl.ds(rs, _RCH), :], a1_ref[pl.ds(rs, _RCH), :])
        p = jax.nn.sigmoid(a - u_n)
        flip = u_b < p
        out_ref[pl.ds(rs, _RCH), pl.ds(cs, _CCH)] = jnp.where(flip, 1.0 - x, x)
        return _

    lax.fori_loop(0, _NRC * _NCC, body, 0, unroll=4)


def kernel(x_start, t):
    cnt = pl.pallas_call(
        _count_zero_kernel,
        grid=(_BATCH // _RB1,),
        in_specs=[pl.BlockSpec((_RB1, _N_ITEMS), lambda i: (i, 0))],
        out_specs=pl.BlockSpec(
            (1, 1), lambda i: (0, 0), memory_space=pltpu.SMEM
        ),
        out_shape=jax.ShapeDtypeStruct((1, 1), jnp.int32),
        compiler_params=pltpu.CompilerParams(
            dimension_semantics=("arbitrary",)
        ),
    )(x_start)

    sparsity = cnt[0, 0].astype(jnp.float32) / jnp.float32(_N_TOTAL)
    gamma_start = 0.1 * (1.0 - sparsity) + 0.001
    gamma_end = gamma_start * 0.1
    epsilon_start = 0.005 * sparsity + 0.0001
    epsilon_end = epsilon_start * 0.1
    gamma = jnp.linspace(gamma_start, gamma_end, _STEPS)
    epsilon = jnp.linspace(epsilon_start, epsilon_end, _STEPS)
    epsilon = jnp.minimum(epsilon, 0.01)
    gamma_cum = 1.0 - jnp.cumprod(1.0 - gamma)
    epsilon_cum = 1.0 - jnp.cumprod(1.0 - epsilon)

    a0 = jnp.take(gamma_cum, t, axis=0)[:, None]
    a1 = jnp.take(epsilon_cum, t, axis=0)[:, None]

    return pl.pallas_call(
        _sample_kernel,
        grid=(_BATCH // _RB2, _GC2),
        in_specs=[
            pl.BlockSpec((_RB2, _CB2), lambda i, j: (i, j)),
            pl.BlockSpec((_RB2, 1), lambda i, j: (i, 0)),
            pl.BlockSpec((_RB2, 1), lambda i, j: (i, 0)),
        ],
        out_specs=pl.BlockSpec((_RB2, _CB2), lambda i, j: (i, j)),
        out_shape=jax.ShapeDtypeStruct((_BATCH, _N_ITEMS), jnp.float32),
        compiler_params=pltpu.CompilerParams(
            dimension_semantics=("parallel", "parallel")
        ),
    )(x_start, a0, a1)


# speedup vs baseline: 1.5878x; 1.0160x over previous
import jax
import jax.numpy as jnp
from jax import lax
from jax.experimental import pallas as pl
from jax.experimental.pallas import tpu as pltpu

_STEPS = 5
_BATCH = 1024
_N_ITEMS = 100000
_N_TOTAL = _BATCH * _N_ITEMS

_NOISE_KEY = (0x85F65B85, 0x97B8C3E1)
_BERN_KEY = (0x181B3F15, 0x67A69C51)

_ROTS = ((13, 15, 26, 6), (17, 29, 16, 24))

_RB1 = 8
_RB2 = 32
_CB2 = 2048
_GC2 = (_N_ITEMS + _CB2 - 1) // _CB2


def _rotl(x, d):
    return lax.shift_left(x, jnp.uint32(d)) | lax.shift_right_logical(
        x, jnp.uint32(32 - d)
    )


def _threefry_bits(key, lo):
    k1 = jnp.uint32(key[0])
    k2 = jnp.uint32(key[1])
    k3 = k1 ^ k2 ^ jnp.uint32(0x1BD11BDA)
    ks = (k1, k2, k3)
    x0 = jnp.full_like(lo, k1)
    x1 = lo + k2
    for g in range(5):
        for r in _ROTS[g % 2]:
            x0 = x0 + x1
            x1 = _rotl(x1, r) ^ x0
        x0 = x0 + ks[(g + 1) % 3]
        x1 = x1 + ks[(g + 2) % 3] + jnp.uint32(g + 1)
    return x0 ^ x1


def _u01(bits):
    fb = lax.shift_right_logical(bits, jnp.uint32(9)) | jnp.uint32(0x3F800000)
    return lax.bitcast_convert_type(fb, jnp.float32) - 1.0


def _count_zero_kernel(x_ref, cnt_ref):
    @pl.when(pl.program_id(0) == 0)
    def _init():
        cnt_ref[0, 0] = jnp.int32(0)

    blk = x_ref[...]
    cnt_ref[0, 0] += jnp.sum((blk == 0.0).astype(jnp.int32))


_RCH = 8
_CCH = 512
_NRC = _RB2 // _RCH
_NCC = _CB2 // _CCH


def _sample_kernel(x_ref, a0_ref, a1_ref, out_ref):
    r0 = pl.program_id(0) * _RB2
    c0 = pl.program_id(1) * _CB2
    rows = lax.broadcasted_iota(jnp.int32, (_RCH, _CCH), 0)
    cols = lax.broadcasted_iota(jnp.int32, (_RCH, _CCH), 1)
    lin00 = ((r0 + rows) * _N_ITEMS + (c0 + cols)).astype(jnp.uint32)

    for rc in range(_NRC):
        for cc in range(_NCC):
            rs = rc * _RCH
            cs = cc * _CCH
            lin = lin00 + jnp.uint32(rs * _N_ITEMS + cs)
            u_b = _u01(_threefry_bits(_BERN_KEY, lin))
            u_n = _u01(_threefry_bits(_NOISE_KEY, lin))
            x = x_ref[rs : rs + _RCH, cs : cs + _CCH]
            a = jnp.where(
                x == 0.0,
                a0_ref[rs : rs + _RCH, :],
                a1_ref[rs : rs + _RCH, :],
            )
            p = jax.nn.sigmoid(a - u_n)
            flip = u_b < p
            out_ref[rs : rs + _RCH, cs : cs + _CCH] = jnp.where(
                flip, 1.0 - x, x
            )


def kernel(x_start, t):
    cnt = pl.pallas_call(
        _count_zero_kernel,
        grid=(_BATCH // _RB1,),
        in_specs=[pl.BlockSpec((_RB1, _N_ITEMS), lambda i: (i, 0))],
        out_specs=pl.BlockSpec(
            (1, 1), lambda i: (0, 0), memory_space=pltpu.SMEM
        ),
        out_shape=jax.ShapeDtypeStruct((1, 1), jnp.int32),
        compiler_params=pltpu.CompilerParams(
            dimension_semantics=("arbitrary",)
        ),
    )(x_start)

    sparsity = cnt[0, 0].astype(jnp.float32) / jnp.float32(_N_TOTAL)
    gamma_start = 0.1 * (1.0 - sparsity) + 0.001
    gamma_end = gamma_start * 0.1
    epsilon_start = 0.005 * sparsity + 0.0001
    epsilon_end = epsilon_start * 0.1
    gamma = jnp.linspace(gamma_start, gamma_end, _STEPS)
    epsilon = jnp.linspace(epsilon_start, epsilon_end, _STEPS)
    epsilon = jnp.minimum(epsilon, 0.01)
    gamma_cum = 1.0 - jnp.cumprod(1.0 - gamma)
    epsilon_cum = 1.0 - jnp.cumprod(1.0 - epsilon)

    a0 = jnp.take(gamma_cum, t, axis=0)[:, None]
    a1 = jnp.take(epsilon_cum, t, axis=0)[:, None]

    return pl.pallas_call(
        _sample_kernel,
        grid=(_BATCH // _RB2, _GC2),
        in_specs=[
            pl.BlockSpec((_RB2, _CB2), lambda i, j: (i, j)),
            pl.BlockSpec((_RB2, 1), lambda i, j: (i, 0)),
            pl.BlockSpec((_RB2, 1), lambda i, j: (i, 0)),
        ],
        out_specs=pl.BlockSpec((_RB2, _CB2), lambda i, j: (i, j)),
        out_shape=jax.ShapeDtypeStruct((_BATCH, _N_ITEMS), jnp.float32),
        compiler_params=pltpu.CompilerParams(
            dimension_semantics=("parallel", "parallel")
        ),
    )(x_start, a0, a1)


# speedup vs baseline: 1.6151x; 1.0172x over previous
import jax
import jax.numpy as jnp
from jax import lax
from jax.experimental import pallas as pl
from jax.experimental.pallas import tpu as pltpu

_STEPS = 5
_BATCH = 1024
_N_ITEMS = 100000
_N_TOTAL = _BATCH * _N_ITEMS

_NOISE_KEY = (0x85F65B85, 0x97B8C3E1)
_BERN_KEY = (0x181B3F15, 0x67A69C51)

_ROTS = ((13, 15, 26, 6), (17, 29, 16, 24))

_RB1 = 8
_RB2 = 128
_CB2 = 2048
_GC2 = (_N_ITEMS + _CB2 - 1) // _CB2


def _rotl(x, d):
    return lax.shift_left(x, jnp.uint32(d)) | lax.shift_right_logical(
        x, jnp.uint32(32 - d)
    )


def _threefry_bits(key, lo):
    k1 = jnp.uint32(key[0])
    k2 = jnp.uint32(key[1])
    k3 = k1 ^ k2 ^ jnp.uint32(0x1BD11BDA)
    ks = (k1, k2, k3)
    x0 = jnp.full_like(lo, k1)
    x1 = lo + k2
    for g in range(5):
        for r in _ROTS[g % 2]:
            x0 = x0 + x1
            x1 = _rotl(x1, r) ^ x0
        x0 = x0 + ks[(g + 1) % 3]
        x1 = x1 + ks[(g + 2) % 3] + jnp.uint32(g + 1)
    return x0 ^ x1


def _u01(bits):
    fb = lax.shift_right_logical(bits, jnp.uint32(9)) | jnp.uint32(0x3F800000)
    return lax.bitcast_convert_type(fb, jnp.float32) - 1.0


def _count_zero_kernel(x_ref, cnt_ref):
    @pl.when(pl.program_id(0) == 0)
    def _init():
        cnt_ref[0, 0] = jnp.int32(0)

    blk = x_ref[...]
    cnt_ref[0, 0] += jnp.sum((blk == 0.0).astype(jnp.int32))


_RCH = 8
_CCH = 512
_NRC = _RB2 // _RCH
_NCC = _CB2 // _CCH


def _sample_kernel(x_ref, a0_ref, a1_ref, out_ref):
    r0 = pl.program_id(0) * _RB2
    c0 = pl.program_id(1) * _CB2
    rows = lax.broadcasted_iota(jnp.int32, (_RCH, _CCH), 0)
    cols = lax.broadcasted_iota(jnp.int32, (_RCH, _CCH), 1)
    lin00 = ((r0 + rows) * _N_ITEMS + (c0 + cols)).astype(jnp.uint32)

    for rc in range(_NRC):
        for cc in range(_NCC):
            rs = rc * _RCH
            cs = cc * _CCH
            lin = lin00 + jnp.uint32(rs * _N_ITEMS + cs)
            u_b = _u01(_threefry_bits(_BERN_KEY, lin))
            u_n = _u01(_threefry_bits(_NOISE_KEY, lin))
            x = x_ref[rs : rs + _RCH, cs : cs + _CCH]
            a = jnp.where(
                x == 0.0,
                a0_ref[rs : rs + _RCH, :],
                a1_ref[rs : rs + _RCH, :],
            )
            p = jax.nn.sigmoid(a - u_n)
            flip = u_b < p
            out_ref[rs : rs + _RCH, cs : cs + _CCH] = jnp.where(
                flip, 1.0 - x, x
            )


def kernel(x_start, t):
    cnt = pl.pallas_call(
        _count_zero_kernel,
        grid=(_BATCH // _RB1,),
        in_specs=[pl.BlockSpec((_RB1, _N_ITEMS), lambda i: (i, 0))],
        out_specs=pl.BlockSpec(
            (1, 1), lambda i: (0, 0), memory_space=pltpu.SMEM
        ),
        out_shape=jax.ShapeDtypeStruct((1, 1), jnp.int32),
        compiler_params=pltpu.CompilerParams(
            dimension_semantics=("arbitrary",)
        ),
    )(x_start)

    sparsity = cnt[0, 0].astype(jnp.float32) / jnp.float32(_N_TOTAL)
    gamma_start = 0.1 * (1.0 - sparsity) + 0.001
    gamma_end = gamma_start * 0.1
    epsilon_start = 0.005 * sparsity + 0.0001
    epsilon_end = epsilon_start * 0.1
    gamma = jnp.linspace(gamma_start, gamma_end, _STEPS)
    epsilon = jnp.linspace(epsilon_start, epsilon_end, _STEPS)
    epsilon = jnp.minimum(epsilon, 0.01)
    gamma_cum = 1.0 - jnp.cumprod(1.0 - gamma)
    epsilon_cum = 1.0 - jnp.cumprod(1.0 - epsilon)

    a0 = jnp.take(gamma_cum, t, axis=0)[:, None]
    a1 = jnp.take(epsilon_cum, t, axis=0)[:, None]

    return pl.pallas_call(
        _sample_kernel,
        grid=(_BATCH // _RB2, _GC2),
        in_specs=[
            pl.BlockSpec((_RB2, _CB2), lambda i, j: (i, j)),
            pl.BlockSpec((_RB2, 1), lambda i, j: (i, 0)),
            pl.BlockSpec((_RB2, 1), lambda i, j: (i, 0)),
        ],
        out_specs=pl.BlockSpec((_RB2, _CB2), lambda i, j: (i, j)),
        out_shape=jax.ShapeDtypeStruct((_BATCH, _N_ITEMS), jnp.float32),
        compiler_params=pltpu.CompilerParams(
            dimension_semantics=("parallel", "parallel")
        ),
    )(x_start, a0, a1)
